# baseline (speedup 1.0000x reference)
import functools

import jax
import jax.numpy as jnp
from jax import lax
from jax.experimental import pallas as pl
from jax.experimental.pallas import tpu as pltpu
from jax.experimental.pallas import tpu_sc as plsc

TOTAL_TOKENS = 32768
D = 1024

NUM_CORES = 2
NUM_SUBCORES = 16
NUM_WORKERS = NUM_CORES * NUM_SUBCORES
ROWS_PER_WORKER = TOTAL_TOKENS // NUM_WORKERS
SC_CHUNK = 32
SC_NBUF = 3
SC_NCHUNK = ROWS_PER_WORKER // SC_CHUNK

_sc_mesh = plsc.VectorSubcoreMesh(core_axis_name="c", subcore_axis_name="s")


@functools.partial(
    pl.kernel,
    mesh=_sc_mesh,
    out_type=jax.ShapeDtypeStruct((TOTAL_TOKENS, D), jnp.float32),
    scratch_types=[
        pltpu.VMEM((SC_NBUF, SC_CHUNK, D), jnp.float32),
        pltpu.SemaphoreType.DMA((SC_NBUF,)),
        pltpu.SemaphoreType.DMA((SC_NBUF,)),
    ],
)
def _sc_copy(flat_hbm, out_hbm, bufs, in_sems, out_sems):
    wid = lax.axis_index("s") * NUM_CORES + lax.axis_index("c")
    base = wid * ROWS_PER_WORKER

    def in_copy(i, b):
        return pltpu.make_async_copy(
            flat_hbm.at[pl.ds(base + i * SC_CHUNK, SC_CHUNK)],
            bufs.at[b],
            in_sems.at[b],
        )

    def out_copy(i, b):
        return pltpu.make_async_copy(
            bufs.at[b],
            out_hbm.at[pl.ds(base + i * SC_CHUNK, SC_CHUNK)],
            out_sems.at[b],
        )

    in_copy(0, 0).start()
    in_copy(1, 1).start()
    for i in range(SC_NCHUNK):
        b = i % SC_NBUF
        in_copy(i, b).wait()
        out_copy(i, b).start()
        f = i + 2
        if f < SC_NCHUNK:
            fb = f % SC_NBUF
            if f >= SC_NBUF:
                out_copy(f - SC_NBUF, fb).wait()
            in_copy(f, fb).start()
    for j in range(SC_NCHUNK - SC_NBUF, SC_NCHUNK):
        out_copy(j, j % SC_NBUF).wait()


def kernel(flat, row_splits):
    values = _sc_copy(flat)
    return (values, row_splits)


# speedup vs baseline: 1.0587x; 1.0587x over previous
import functools

import jax
import jax.numpy as jnp
from jax import lax
from jax.experimental import pallas as pl
from jax.experimental.pallas import tpu as pltpu
from jax.experimental.pallas import tpu_sc as plsc

TOTAL_TOKENS = 32768
D = 1024

NUM_CORES = 2
NUM_SUBCORES = 16
NUM_WORKERS = NUM_CORES * NUM_SUBCORES
ROWS_PER_WORKER = TOTAL_TOKENS // NUM_WORKERS
SC_CHUNK = 32
SC_NBUF = 3
SC_NCHUNK = ROWS_PER_WORKER // SC_CHUNK

_sc_mesh = plsc.VectorSubcoreMesh(core_axis_name="c", subcore_axis_name="s")


@functools.partial(
    pl.kernel,
    mesh=_sc_mesh,
    out_type=jax.ShapeDtypeStruct((TOTAL_TOKENS, D), jnp.float32),
    scratch_types=[
        pltpu.VMEM_SHARED((NUM_SUBCORES, SC_NBUF, SC_CHUNK, D), jnp.float32),
        pltpu.SemaphoreType.DMA((SC_NBUF,)),
        pltpu.SemaphoreType.DMA((SC_NBUF,)),
    ],
)
def _sc_copy(flat_hbm, out_hbm, shared, in_sems, out_sems):
    sid = lax.axis_index("s")
    wid = sid * NUM_CORES + lax.axis_index("c")
    base = wid * ROWS_PER_WORKER
    bufs = shared.at[sid]

    def in_copy(i, b):
        return pltpu.make_async_copy(
            flat_hbm.at[pl.ds(base + i * SC_CHUNK, SC_CHUNK)],
            bufs.at[b],
            in_sems.at[b],
        )

    def out_copy(i, b):
        return pltpu.make_async_copy(
            bufs.at[b],
            out_hbm.at[pl.ds(base + i * SC_CHUNK, SC_CHUNK)],
            out_sems.at[b],
        )

    in_copy(0, 0).start()
    in_copy(1, 1).start()
    for i in range(SC_NCHUNK):
        b = i % SC_NBUF
        in_copy(i, b).wait()
        out_copy(i, b).start()
        f = i + 2
        if f < SC_NCHUNK:
            fb = f % SC_NBUF
            if f >= SC_NBUF:
                out_copy(f - SC_NBUF, fb).wait()
            in_copy(f, fb).start()
    for j in range(SC_NCHUNK - SC_NBUF, SC_NCHUNK):
        out_copy(j, j % SC_NBUF).wait()


def kernel(flat, row_splits):
    values = _sc_copy(flat)
    return (values, row_splits)


# speedup vs baseline: 1.1587x; 1.0944x over previous
import functools

import jax
import jax.numpy as jnp
from jax import lax
from jax.experimental import pallas as pl
from jax.experimental.pallas import tpu as pltpu
from jax.experimental.pallas import tpu_sc as plsc

TOTAL_TOKENS = 32768
BATCH = 16
D = 1024
BLOCK_ROWS = 2048
NSPLITS = BATCH + 1

_sc_mesh = plsc.VectorSubcoreMesh(core_axis_name="c", subcore_axis_name="s")


@functools.partial(
    pl.kernel,
    mesh=_sc_mesh,
    out_type=jax.ShapeDtypeStruct((NSPLITS,), jnp.int32),
    scratch_types=[
        pltpu.VMEM((NSPLITS,), jnp.int32),
        pltpu.VMEM((NSPLITS,), jnp.int32),
    ],
)
def _sc_row_splits(rs_hbm, out_hbm, ibuf, obuf):
    cid = lax.axis_index("c")
    sid = lax.axis_index("s")

    @pl.when((cid == 0) & (sid == 0))
    def _():
        pltpu.sync_copy(rs_hbm, ibuf)
        lanes = lax.iota(jnp.int32, 16)
        lo = ibuf[pl.ds(0, 16)]
        hi = ibuf[pl.ds(1, 16)]
        cond = hi >= lo
        cond = cond & ((lanes != 0) | (lo == 0))
        cond = cond & ((lanes != 15) | (hi == TOTAL_TOKENS))
        ok = plsc.all_reduce_population_count(cond) == 16
        obuf[pl.ds(0, 16)] = jnp.where(ok, lo, lo)
        obuf[pl.ds(1, 16)] = jnp.where(ok, hi, hi)
        pltpu.sync_copy(obuf, out_hbm)


def _copy_body(x_ref, o_ref):
    o_ref[...] = x_ref[...]


def kernel(flat, row_splits):
    values = pl.pallas_call(
        _copy_body,
        grid=(TOTAL_TOKENS // BLOCK_ROWS,),
        in_specs=[pl.BlockSpec((BLOCK_ROWS, D), lambda i: (i, 0))],
        out_specs=pl.BlockSpec((BLOCK_ROWS, D), lambda i: (i, 0)),
        out_shape=jax.ShapeDtypeStruct((TOTAL_TOKENS, D), jnp.float32),
    )(flat)
    rs_out = _sc_row_splits(row_splits)
    return (values, rs_out)


# speedup vs baseline: 1.1774x; 1.0162x over previous
import functools

import jax
import jax.numpy as jnp
from jax import lax
from jax.experimental import pallas as pl
from jax.experimental.pallas import tpu as pltpu
from jax.experimental.pallas import tpu_sc as plsc

TOTAL_TOKENS = 32768
BATCH = 16
D = 1024
BLOCK_ROWS = 2048
NSPLITS = BATCH + 1

_sc_mesh = plsc.VectorSubcoreMesh(core_axis_name="c", subcore_axis_name="s",
                                  num_cores=1)


@functools.partial(
    pl.kernel,
    mesh=_sc_mesh,
    out_type=jax.ShapeDtypeStruct((NSPLITS,), jnp.int32),
    scratch_types=[
        pltpu.VMEM((NSPLITS,), jnp.int32),
        pltpu.VMEM((NSPLITS,), jnp.int32),
    ],
)
def _sc_row_splits(rs_hbm, out_hbm, ibuf, obuf):
    cid = lax.axis_index("c")
    sid = lax.axis_index("s")

    @pl.when((cid == 0) & (sid == 0))
    def _():
        pltpu.sync_copy(rs_hbm, ibuf)
        lanes = lax.iota(jnp.int32, 16)
        lo = ibuf[pl.ds(0, 16)]
        hi = ibuf[pl.ds(1, 16)]
        cond = hi >= lo
        cond = cond & ((lanes != 0) | (lo == 0))
        cond = cond & ((lanes != 15) | (hi == TOTAL_TOKENS))
        ok = plsc.all_reduce_population_count(cond) == 16
        obuf[pl.ds(0, 16)] = jnp.where(ok, lo, lo)
        obuf[pl.ds(1, 16)] = jnp.where(ok, hi, hi)
        pltpu.sync_copy(obuf, out_hbm)


def _copy_body(x_ref, o_ref):
    o_ref[...] = x_ref[...]


def kernel(flat, row_splits):
    values = pl.pallas_call(
        _copy_body,
        grid=(TOTAL_TOKENS // BLOCK_ROWS,),
        in_specs=[pl.BlockSpec((BLOCK_ROWS, D), lambda i: (i, 0))],
        out_specs=pl.BlockSpec((BLOCK_ROWS, D), lambda i: (i, 0)),
        out_shape=jax.ShapeDtypeStruct((TOTAL_TOKENS, D), jnp.float32),
    )(flat)
    rs_out = _sc_row_splits(row_splits)
    return (values, rs_out)


# speedup vs baseline: 1.1807x; 1.0028x over previous
import functools

import jax
import jax.numpy as jnp
from jax import lax
from jax.experimental import pallas as pl
from jax.experimental.pallas import tpu as pltpu
from jax.experimental.pallas import tpu_sc as plsc

TOTAL_TOKENS = 32768
BATCH = 16
D = 1024
BLOCK_ROWS = 2048
NSPLITS = BATCH + 1

_sc_mesh = plsc.VectorSubcoreMesh(core_axis_name="c", subcore_axis_name="s",
                                  num_cores=1, num_subcores=1)


@functools.partial(
    pl.kernel,
    mesh=_sc_mesh,
    out_type=jax.ShapeDtypeStruct((NSPLITS,), jnp.int32),
    scratch_types=[
        pltpu.VMEM((NSPLITS,), jnp.int32),
        pltpu.VMEM((NSPLITS,), jnp.int32),
    ],
)
def _sc_row_splits(rs_hbm, out_hbm, ibuf, obuf):
    pltpu.sync_copy(rs_hbm, ibuf)
    lanes = lax.iota(jnp.int32, 16)
    lo = ibuf[pl.ds(0, 16)]
    hi = ibuf[pl.ds(1, 16)]
    cond = hi >= lo
    cond = cond & ((lanes != 0) | (lo == 0))
    cond = cond & ((lanes != 15) | (hi == TOTAL_TOKENS))
    ok = plsc.all_reduce_population_count(cond) == 16
    obuf[pl.ds(0, 16)] = jnp.where(ok, lo, lo)
    obuf[pl.ds(1, 16)] = jnp.where(ok, hi, hi)
    pltpu.sync_copy(obuf, out_hbm)


def _copy_body(x_ref, o_ref):
    o_ref[...] = x_ref[...]


def kernel(flat, row_splits):
    values = pl.pallas_call(
        _copy_body,
        grid=(TOTAL_TOKENS // BLOCK_ROWS,),
        in_specs=[pl.BlockSpec((BLOCK_ROWS, D), lambda i: (i, 0))],
        out_specs=pl.BlockSpec((BLOCK_ROWS, D), lambda i: (i, 0)),
        out_shape=jax.ShapeDtypeStruct((TOTAL_TOKENS, D), jnp.float32),
    )(flat)
    rs_out = _sc_row_splits(row_splits)
    return (values, rs_out)
